# baseline (speedup 1.0000x reference)
import functools

import jax
import jax.numpy as jnp
from jax import lax
from jax.experimental import pallas as pl
from jax.experimental.pallas import tpu as pltpu
from jax.experimental.pallas import tpu_sc as plsc

_N, _C, _K = 4, 8192, 2048
_IN_F, _OUT_F = 128, 128
_LANES = 16
_NWORKERS = 32
_MASK_LEN = _N * _C
_SLICE = _MASK_LEN // _NWORKERS
_NIDX = _N * _K


_UNROLL = 4


def _sc_mask_kernel(idx_hbm, mask_hbm, idx_v, buf_v):
    cid = lax.axis_index("c")
    sid = lax.axis_index("s")
    wid = sid * 2 + cid
    n = wid // 8
    cbase = (wid % 8) * _SLICE

    pltpu.sync_copy(idx_hbm.at[pl.ds(n * _K, _K)], idx_v)

    zeros = jnp.zeros((_LANES,), jnp.float32)
    ones = jnp.ones((_LANES,), jnp.float32)

    def zero_body(i, carry):
        buf_v[pl.ds(i * _LANES, _LANES)] = zeros
        return carry

    lax.fori_loop(0, _SLICE // _LANES, zero_body, 0)

    def scan_body(j, carry):
        for u in range(_UNROLL):
            v = idx_v[pl.ds((j * _UNROLL + u) * _LANES, _LANES)]
            m = (v >= cbase) & (v < cbase + _SLICE)
            lidx = jnp.where(m, v - cbase, 0)
            plsc.store_scatter(buf_v, [lidx], ones, mask=m)
        return carry

    lax.fori_loop(0, _K // (_LANES * _UNROLL), scan_body, 0)

    pltpu.sync_copy(buf_v, mask_hbm.at[pl.ds(wid * _SLICE, _SLICE)])


def _build_mask(idx_flat):
    mesh = plsc.VectorSubcoreMesh(core_axis_name="c", subcore_axis_name="s")
    return pl.kernel(
        _sc_mask_kernel,
        mesh=mesh,
        compiler_params=pltpu.CompilerParams(needs_layout_passes=False),
        out_type=jax.ShapeDtypeStruct((_MASK_LEN,), jnp.float32),
        scratch_types=[
            pltpu.VMEM((_K,), jnp.int32),
            pltpu.VMEM((_SLICE,), jnp.float32),
        ],
    )(idx_flat)


_BLK = 32768


def _tc_linear_kernel(x_ref, m_ref, w_ref, b_ref, o_ref):
    y = jax.lax.dot_general(
        x_ref[...],
        w_ref[...],
        (((1,), (1,)), ((), ())),
        preferred_element_type=jnp.float32,
    )
    m = m_ref[...].reshape(_BLK, 1)
    o_ref[...] = jnp.where(m > 0.0, y + b_ref[...], 0.0)


def _masked_linear(x2d, mask, w, bias2d):
    rows = x2d.shape[0]
    return pl.pallas_call(
        _tc_linear_kernel,
        grid=(rows // _BLK,),
        in_specs=[
            pl.BlockSpec((_BLK, _IN_F), lambda i: (i, 0)),
            pl.BlockSpec((_BLK,), lambda i: (i,)),
            pl.BlockSpec((_OUT_F, _IN_F), lambda i: (0, 0)),
            pl.BlockSpec((1, _OUT_F), lambda i: (0, 0)),
        ],
        out_specs=pl.BlockSpec((_BLK, _OUT_F), lambda i: (i, 0)),
        out_shape=jax.ShapeDtypeStruct((rows, _OUT_F), jnp.float32),
    )(x2d, mask, w, bias2d)


@jax.jit
def kernel(input, channel_indices, weight, bias):
    n, c, h = input.shape
    idx_flat = channel_indices.reshape(n * channel_indices.shape[1])
    mask = _build_mask(idx_flat)
    out2d = _masked_linear(
        input.reshape(n * c, h),
        mask,
        weight,
        bias.reshape(1, _OUT_F),
    )
    return out2d.reshape(n, c, _OUT_F)


# speedup vs baseline: 1.1233x; 1.1233x over previous
import functools

import jax
import jax.numpy as jnp
from jax import lax
from jax.experimental import pallas as pl
from jax.experimental.pallas import tpu as pltpu
from jax.experimental.pallas import tpu_sc as plsc

_N, _C, _K = 4, 8192, 2048
_IN_F, _OUT_F = 128, 128
_LANES = 16
_NWORKERS = 32
_MASK_LEN = _N * _C
_SLICE = _MASK_LEN // _NWORKERS
_NIDX = _N * _K


_UNROLL = 4


def _sc_mask_kernel(idx_hbm, mask_hbm, idx_v, buf_v):
    cid = lax.axis_index("c")
    sid = lax.axis_index("s")
    wid = sid * 2 + cid
    n = wid // 8
    cbase = (wid % 8) * _SLICE

    pltpu.sync_copy(idx_hbm.at[pl.ds(n * _K, _K)], idx_v)

    zeros = jnp.zeros((_LANES,), jnp.float32)
    ones = jnp.ones((_LANES,), jnp.float32)

    def zero_body(i, carry):
        buf_v[pl.ds(i * _LANES, _LANES)] = zeros
        return carry

    lax.fori_loop(0, _SLICE // _LANES, zero_body, 0)

    def scan_body(j, carry):
        for u in range(_UNROLL):
            v = idx_v[pl.ds((j * _UNROLL + u) * _LANES, _LANES)]
            m = (v >= cbase) & (v < cbase + _SLICE)
            lidx = jnp.where(m, v - cbase, 0)
            plsc.store_scatter(buf_v, [lidx], ones, mask=m)
        return carry

    lax.fori_loop(0, _K // (_LANES * _UNROLL), scan_body, 0)

    pltpu.sync_copy(buf_v, mask_hbm.at[pl.ds(wid * _SLICE, _SLICE)])


def _build_mask(idx_flat):
    mesh = plsc.VectorSubcoreMesh(core_axis_name="c", subcore_axis_name="s")
    return pl.kernel(
        _sc_mask_kernel,
        mesh=mesh,
        compiler_params=pltpu.CompilerParams(needs_layout_passes=False),
        out_type=jax.ShapeDtypeStruct((_MASK_LEN,), jnp.float32),
        scratch_types=[
            pltpu.VMEM((_K,), jnp.int32),
            pltpu.VMEM((_SLICE,), jnp.float32),
        ],
    )(idx_flat)


_BLK = 16384


def _tc_linear_kernel(x_ref, m_ref, w_ref, b_ref, o_ref):
    y = jax.lax.dot_general(
        x_ref[...],
        w_ref[...],
        (((1,), (1,)), ((), ())),
        preferred_element_type=jnp.float32,
    )
    m = m_ref[...].reshape(_BLK, 1)
    o_ref[...] = jnp.where(m > 0.0, y + b_ref[...], 0.0)


def _masked_linear(x2d, mask, w, bias2d):
    rows = x2d.shape[0]
    return pl.pallas_call(
        _tc_linear_kernel,
        grid=(rows // _BLK,),
        in_specs=[
            pl.BlockSpec((_BLK, _IN_F), lambda i: (i, 0)),
            pl.BlockSpec((_BLK,), lambda i: (i,)),
            pl.BlockSpec((_OUT_F, _IN_F), lambda i: (0, 0)),
            pl.BlockSpec((1, _OUT_F), lambda i: (0, 0)),
        ],
        out_specs=pl.BlockSpec((_BLK, _OUT_F), lambda i: (i, 0)),
        out_shape=jax.ShapeDtypeStruct((rows, _OUT_F), jnp.float32),
    )(x2d, mask, w, bias2d)


@jax.jit
def kernel(input, channel_indices, weight, bias):
    n, c, h = input.shape
    idx_flat = channel_indices.reshape(n * channel_indices.shape[1])
    mask = _build_mask(idx_flat)
    out2d = _masked_linear(
        input.reshape(n * c, h),
        mask,
        weight,
        bias.reshape(1, _OUT_F),
    )
    return out2d.reshape(n, c, _OUT_F)
